# baseline (speedup 1.0000x reference)
import jax
import jax.numpy as jnp
from jax.experimental import pallas as pl


def _pna_j(x, src, dst, n, W, b):
    deg = jnp.bincount(dst, length=n).astype(x.dtype)
    delta = jnp.mean(jnp.log(deg + 1.0))
    m = x[src]
    degc = jnp.maximum(deg, 1.0)[:, None]
    s = jax.ops.segment_sum(m, dst, num_segments=n)
    mean = s / degc
    sq = jax.ops.segment_sum(m * m, dst, num_segments=n)
    std = jnp.sqrt(jnp.maximum(sq / degc - mean ** 2, 0.0) + 1e-5)
    mx = jax.ops.segment_max(m, dst, num_segments=n)
    mn = jax.ops.segment_min(m, dst, num_segments=n)
    has = (deg > 0)[:, None]
    mx = jnp.where(has, mx, 0.0)
    mn = jnp.where(has, mn, 0.0)
    aggs = jnp.concatenate([mean, mn, mx, std], axis=1)
    logd = jnp.log(deg + 1.0)[:, None]
    amp = logd / delta
    att = delta / jnp.maximum(logd, 1e-5)
    h = jnp.concatenate([aggs, aggs * amp, aggs * att], axis=1)
    return h @ W + b


def _bn_j(h, g, b):
    mu = jnp.mean(h, axis=0)
    var = jnp.var(h, axis=0)
    return g * (h - mu) / jnp.sqrt(var + 1e-5) + b


def _id_kernel(x_ref, o_ref):
    o_ref[...] = x_ref[...]


def kernel(x, edge_index, W1, b1, g1, be1, W2, b2, g2, be2, W3, b3, g3, be3, W4, b4, g4, be4, Wc, bc):
    src, dst = edge_index[0], edge_index[1]
    n = x.shape[0]
    h = _pna_j(x, src, dst, n, W1, b1)
    h = _bn_j(jax.nn.relu(h), g1, be1)
    h = _pna_j(h, src, dst, n, W2, b2)
    h = _bn_j(jax.nn.relu(h), g2, be2)
    h = _pna_j(h, src, dst, n, W3, b3)
    h = _bn_j(jax.nn.relu(h), g3, be3)
    h = _pna_j(h, src, dst, n, W4, b4)
    h = _bn_j(jax.nn.relu(h), g4, be4)
    out = h @ Wc + bc
    return pl.pallas_call(
        _id_kernel,
        out_shape=jax.ShapeDtypeStruct(out.shape, out.dtype),
    )(out)


# speedup vs baseline: 1.8474x; 1.8474x over previous
import functools

import jax
import jax.numpy as jnp
from jax import lax
from jax.experimental import pallas as pl
from jax.experimental.pallas import tpu as pltpu
from jax.experimental.pallas import tpu_sc as plsc

N = 10000
E = 320000
NOWN = 128
RPO = 79
STRIDE = 80
NW = 32
NPAD = NOWN * RPO
NP80 = NOWN * STRIDE
ECAP = E + NOWN * 128 + 128
NCH_E = E // 128
F32MAX = 3.4e38

_ioti = lambda: lax.iota(jnp.int32, 16)


def _div79(v):
    return lax.shift_right_logical(v * 53093, 22)


def _sx(vref, idx):
    return vref[pl.ds(idx, 16)][0]


def _lift16(scalars):
    v = jnp.zeros((16,), jnp.int32)
    it = _ioti()
    for l, s in enumerate(scalars):
        v = jnp.where(it == l, s, v)
    return v


def _scmesh():
    return plsc.VectorSubcoreMesh(core_axis_name="c", subcore_axis_name="s")


def _wid():
    return lax.axis_index("s") * 2 + lax.axis_index("c")


def _make_hist():
    @functools.partial(
        pl.kernel, mesh=_scmesh(),
        out_type=jax.ShapeDtypeStruct((NW, NOWN), jnp.int32),
        scratch_types=[
            pltpu.VMEM((128,), jnp.int32),
            pltpu.VMEM((128,), jnp.int32),
            pltpu.SMEM((128,), jnp.int32),
        ],
    )
    def hist(dst_hbm, counts_hbm, dbuf, cntv, cnt_s):
        w = _wid()

        def zi(i, c):
            cnt_s[i] = 0
            return c
        lax.fori_loop(0, 128, zi, 0)

        nch = jnp.where(w < NCH_E % NW, NCH_E // NW + 1, NCH_E // NW)

        def chunk(ci, c):
            base = pl.multiple_of((w + NW * ci) * 128, 128)
            pltpu.sync_copy(dst_hbm.at[pl.ds(base, 128)], dbuf.at[pl.ds(0, 128)])

            def kb(k, c2):
                dv = dbuf[pl.ds(pl.multiple_of(16 * k, 16), 16)]
                for l in range(16):
                    o = dv[l] // RPO
                    cnt_s[o] = cnt_s[o] + 1
                return c2
            return lax.fori_loop(0, 8, kb, c)
        lax.fori_loop(0, nch, chunk, 0)

        def wv(j, c):
            return c
        for j in range(8):
            cntv[pl.ds(16 * j, 16)] = _lift16([cnt_s[16 * j + l] for l in range(16)])
        pltpu.sync_copy(cntv, counts_hbm.at[w])

    return hist


def _scan_body(cnt_ref, s_ref, meta_ref):
    cnt = cnt_ref[...]
    tot = jnp.sum(cnt, axis=0, keepdims=True)
    pt = ((tot + 127) // 128) * 128
    cpt = pt
    for k in (1, 2, 4, 8, 16, 32, 64):
        cpt = cpt + jnp.concatenate(
            [jnp.zeros((1, k), jnp.int32), cpt[:, :-k]], axis=1)
    ob = cpt - pt
    sumpt = jnp.max(cpt, axis=1, keepdims=True)
    cw = cnt
    for k in (1, 2, 4, 8, 16):
        cw = cw + jnp.concatenate(
            [jnp.zeros((k, NOWN), jnp.int32), cw[:-k, :]], axis=0)
    s_ref[...] = ob + cw - cnt
    meta = jnp.concatenate(
        [ob, tot, pt, jnp.broadcast_to(sumpt, (1, NOWN)),
         jnp.zeros((4, NOWN), jnp.int32)], axis=0)
    meta_ref[...] = meta


def _scan(counts):
    return pl.pallas_call(
        _scan_body,
        out_shape=[
            jax.ShapeDtypeStruct((NW, NOWN), jnp.int32),
            jax.ShapeDtypeStruct((8, NOWN), jnp.int32),
        ],
    )(counts)


def _make_scatter():
    @functools.partial(
        pl.kernel, mesh=_scmesh(),
        out_type=[
            jax.ShapeDtypeStruct((ECAP,), jnp.int32),
            jax.ShapeDtypeStruct((ECAP,), jnp.int32),
        ],
        scratch_types=[
            pltpu.VMEM((128,), jnp.int32),
            pltpu.VMEM((144,), jnp.int32),
            pltpu.VMEM((128,), jnp.int32),
            pltpu.VMEM((128,), jnp.int32),
            pltpu.VMEM((128,), jnp.int32),
            pltpu.VMEM((144,), jnp.int32),
            pltpu.VMEM((144,), jnp.int32),
            pltpu.VMEM((144,), jnp.int32),
            pltpu.VMEM((144,), jnp.int32),
            pltpu.VMEM((128,), jnp.int32),
            pltpu.VMEM((128,), jnp.int32),
            pltpu.SMEM((128,), jnp.int32),
            pltpu.SemaphoreType.DMA,
            pltpu.SemaphoreType.DMA,
        ],
    )
    def scat(src_hbm, dst_hbm, s_hbm, meta_hbm, ssrc_hbm, sdloc_hbm,
             sbuf, dbuf, posv, dlocv, srow, obv, totv, ptv, spv, zb, nb,
             start_s, sem1, sem2):
        w = _wid()
        it = _ioti()
        pltpu.sync_copy(s_hbm.at[w], srow)
        pltpu.sync_copy(meta_hbm.at[0], obv.at[pl.ds(0, 128)])
        pltpu.sync_copy(meta_hbm.at[1], totv.at[pl.ds(0, 128)])
        pltpu.sync_copy(meta_hbm.at[2], ptv.at[pl.ds(0, 128)])
        pltpu.sync_copy(meta_hbm.at[3], spv.at[pl.ds(0, 128)])
        for j in range(8):
            sv = srow[pl.ds(16 * j, 16)]
            for l in range(16):
                start_s[16 * j + l] = sv[l]
        z16 = jnp.zeros((16,), jnp.int32)
        n16 = jnp.full((16,), RPO, jnp.int32)
        for j in range(8):
            zb[pl.ds(16 * j, 16)] = z16
            nb[pl.ds(16 * j, 16)] = n16

        dump = jnp.int32(ECAP - NW) + w

        nch = jnp.where(w < NCH_E % NW, NCH_E // NW + 1, NCH_E // NW)

        def chunk(ci, c):
            base = pl.multiple_of((w + NW * ci) * 128, 128)
            pltpu.sync_copy(src_hbm.at[pl.ds(base, 128)], sbuf)
            pltpu.sync_copy(dst_hbm.at[pl.ds(base, 128)], dbuf.at[pl.ds(0, 128)])

            def kb(k, c2):
                sl = pl.ds(pl.multiple_of(16 * k, 16), 16)
                sv = sbuf[sl]
                sbuf[sl] = sv + _div79(sv)
                dv = dbuf[sl]
                dlocv[sl] = dv - _div79(dv) * RPO

                def lb(l, pos):
                    o = dbuf[pl.ds(16 * k + l, 16)][0] // RPO
                    p = start_s[o]
                    start_s[o] = p + 1
                    return jnp.where(_ioti() == l, p, pos)
                posv[sl] = lax.fori_loop(0, 16, lb, jnp.zeros((16,), jnp.int32))
                return c2
            lax.fori_loop(0, 8, kb, c)
            pltpu.async_copy(sbuf, ssrc_hbm.at[posv], sem1).wait()
            pltpu.async_copy(dlocv, sdloc_hbm.at[posv], sem2).wait()
            return c
        lax.fori_loop(0, nch, chunk, 0)

        for og in range(4):
            o = 4 * w + og
            ob_o = _sx(obv, o)
            tot_o = _sx(totv, o)
            pt_o = _sx(ptv, o)
            lim = ob_o + pt_o
            for j in range(8):
                p = ob_o + tot_o + it + 16 * j
                posv[pl.ds(16 * j, 16)] = jnp.where(p < lim, p, dump)
            pltpu.async_copy(zb, ssrc_hbm.at[posv], sem1).wait()
            pltpu.async_copy(nb, sdloc_hbm.at[posv], sem2).wait()

        sumpt = _sx(spv, jnp.int32(0))
        tail_lim = jnp.int32(ECAP - NW)

        def tl(t, c):
            base = sumpt + 128 * (w + NW * t)
            for j in range(8):
                p = base + it + 16 * j
                posv[pl.ds(16 * j, 16)] = jnp.where(p < tail_lim, p, dump)
            pltpu.async_copy(zb, ssrc_hbm.at[posv], sem1).wait()
            pltpu.async_copy(nb, sdloc_hbm.at[posv], sem2).wait()
            return c
        lax.fori_loop(0, 5, tl, 0)

        for j in range(8):
            posv[pl.ds(16 * j, 16)] = jnp.full((16,), dump, jnp.int32)
        pltpu.async_copy(zb, ssrc_hbm.at[posv], sem1).wait()
        pltpu.async_copy(nb, sdloc_hbm.at[posv], sem2).wait()

    return scat


def _make_deg():
    @functools.partial(
        pl.kernel, mesh=_scmesh(),
        out_type=jax.ShapeDtypeStruct((NP80, 128), jnp.float32),
        scratch_types=[
            pltpu.VMEM((128,), jnp.int32),
            pltpu.VMEM((STRIDE, 128), jnp.float32),
            pltpu.VMEM((144,), jnp.int32),
            pltpu.VMEM((144,), jnp.int32),
            pltpu.SMEM((STRIDE,), jnp.float32),
        ],
    )
    def deg(sdloc_hbm, meta_hbm, degb_hbm, dbuf, rowbuf, obv, ptv, cnt_s):
        w = _wid()
        pltpu.sync_copy(meta_hbm.at[0], obv.at[pl.ds(0, 128)])
        pltpu.sync_copy(meta_hbm.at[2], ptv.at[pl.ds(0, 128)])
        for og in range(4):
            o = 4 * w + og
            ob_o = _sx(obv, o)
            pt_o = _sx(ptv, o)

            def zi(i, c):
                cnt_s[i] = 0.0
                return c
            lax.fori_loop(0, STRIDE, zi, 0)

            def chunk(ci, c):
                pltpu.sync_copy(sdloc_hbm.at[pl.ds(pl.multiple_of(ob_o + 128 * ci, 128), 128)], dbuf)

                def kb(k, c2):
                    dv = dbuf[pl.ds(pl.multiple_of(16 * k, 16), 16)]
                    for l in range(16):
                        d = dv[l]
                        cnt_s[d] = cnt_s[d] + 1.0
                    return c2
                return lax.fori_loop(0, 8, kb, c)
            lax.fori_loop(0, pt_o // 128, chunk, 0)

            def wrow(r, c):
                v = jnp.full((16,), cnt_s[r], jnp.float32)
                for f in range(8):
                    rowbuf[r, pl.ds(16 * f, 16)] = v
                return c
            lax.fori_loop(0, STRIDE, wrow, 0)
            pltpu.sync_copy(rowbuf,
                            degb_hbm.at[pl.ds(pl.multiple_of(STRIDE * o, 8), STRIDE)])

    return deg


_BLK = 640


def _valid(i, b, h):
    g = lax.broadcasted_iota(jnp.int32, (b, h), 0) + i * b
    own = g // STRIDE
    rloc = g - own * STRIDE
    node = own * RPO + rloc
    return (rloc < RPO) & (node < N)


def _prep_body(degb_ref, delta_ref):
    i = pl.program_id(0)
    d = degb_ref[...]
    val = jnp.where(_valid(i, _BLK, 128), jnp.log(d + 1.0), 0.0)
    ps = jnp.sum(val, axis=0, keepdims=True) / N

    @pl.when(i == 0)
    def _():
        delta_ref[...] = jnp.zeros((8, 128), jnp.float32)
    delta_ref[0:1, :] += ps


def _prep(degb):
    return pl.pallas_call(
        _prep_body,
        grid=(NP80 // _BLK,),
        in_specs=[pl.BlockSpec((_BLK, 128), lambda i: (i, 0))],
        out_specs=pl.BlockSpec((8, 128), lambda i: (0, 0)),
        out_shape=jax.ShapeDtypeStruct((8, 128), jnp.float32),
    )(degb)


def _make_reduce(tn, f, opa, ch):
    rows_acc = STRIDE * opa
    rounds = 4 // opa
    nf = f // 16

    @functools.partial(
        pl.kernel, mesh=_scmesh(),
        out_type=[jax.ShapeDtypeStruct((NP80, f), jnp.float32)
                  for _ in range(4)],
        scratch_types=[
            pltpu.VMEM((ch,), jnp.int32),
            pltpu.VMEM((ch + 16,), jnp.int32),
            pltpu.VMEM((ch, f), jnp.float32),
            pltpu.VMEM((rows_acc, f), jnp.float32),
            pltpu.VMEM((rows_acc, f), jnp.float32),
            pltpu.VMEM((rows_acc, f), jnp.float32),
            pltpu.VMEM((rows_acc, f), jnp.float32),
            pltpu.VMEM((144,), jnp.int32),
            pltpu.VMEM((144,), jnp.int32),
            pltpu.SemaphoreType.DMA,
        ],
    )
    def red(table_hbm, ssrc_hbm, sdloc_hbm, meta_hbm,
            os_hbm, oq_hbm, om_hbm, on_hbm,
            sidx, dlbuf, rows, acc_s, acc_q, acc_m, acc_n, obv, ptv, gsem):
        w = _wid()
        pltpu.sync_copy(meta_hbm.at[0], obv.at[pl.ds(0, 128)])
        pltpu.sync_copy(meta_hbm.at[2], ptv.at[pl.ds(0, 128)])
        z16 = jnp.zeros((16,), jnp.float32)
        lo16 = jnp.full((16,), -F32MAX, jnp.float32)
        hi16 = jnp.full((16,), F32MAX, jnp.float32)

        for rnd in range(rounds):
            def zrow(r, c):
                for fi in range(nf):
                    sl = pl.ds(16 * fi, 16)
                    acc_s[r, sl] = z16
                    acc_q[r, sl] = z16
                    acc_m[r, sl] = lo16
                    acc_n[r, sl] = hi16
                return c
            lax.fori_loop(0, rows_acc, zrow, 0)

            for seg in range(opa):
                o = 4 * w + rnd * opa + seg
                ob_o = _sx(obv, o)
                pt_o = _sx(ptv, o)
                rbase = STRIDE * seg

                def chunk(ci, c):
                    base = pl.multiple_of(ob_o + ch * ci, ch)
                    pltpu.sync_copy(ssrc_hbm.at[pl.ds(base, ch)], sidx)
                    pltpu.sync_copy(sdloc_hbm.at[pl.ds(base, ch)], dlbuf.at[pl.ds(0, ch)])
                    pltpu.async_copy(table_hbm.at[sidx], rows, gsem).wait()

                    def eb(e, c2):
                        row = rbase + dlbuf[pl.ds(e, 16)][0]
                        for fi in range(nf):
                            sl = pl.ds(16 * fi, 16)
                            r = rows[e, sl]
                            acc_s[row, sl] += r
                            acc_q[row, sl] += r * r
                            acc_m[row, sl] = jnp.maximum(acc_m[row, sl], r)
                            acc_n[row, sl] = jnp.minimum(acc_n[row, sl], r)
                        return c2
                    return lax.fori_loop(0, ch, eb, c)
                lax.fori_loop(0, pt_o // ch, chunk, 0)

                dst_rows = pl.ds(pl.multiple_of(STRIDE * o, 8), STRIDE)
                src_rows = pl.ds(rbase, STRIDE)
                pltpu.sync_copy(acc_s.at[src_rows], os_hbm.at[dst_rows])
                pltpu.sync_copy(acc_q.at[src_rows], oq_hbm.at[dst_rows])
                pltpu.sync_copy(acc_m.at[src_rows], om_hbm.at[dst_rows])
                pltpu.sync_copy(acc_n.at[src_rows], on_hbm.at[dst_rows])

    return red


def _xf(v, f):
    if f == 128:
        return v
    return jnp.concatenate([v] * (f // 128), axis=1)


def _make_layer(f, h):
    def body(s_ref, q_ref, mx_ref, mn_ref, degb_ref, delta_ref, w_ref, b_ref,
             y_ref, st_ref):
        i = pl.program_id(0)
        deg = degb_ref[...]
        dl = delta_ref[0:1, :]
        degc = jnp.maximum(deg, 1.0)
        logd = jnp.log(deg + 1.0)
        amp = logd / dl
        att = dl / jnp.maximum(logd, 1e-5)
        has = deg > 0.0

        degc_x = _xf(degc, f)
        has_x = _xf(has, f)
        s = s_ref[...]
        q = q_ref[...]
        mean = s / degc_x
        std = jnp.sqrt(jnp.maximum(q / degc_x - mean * mean, 0.0) + 1e-5)
        mx = jnp.where(has_x, mx_ref[...], 0.0)
        mn = jnp.where(has_x, mn_ref[...], 0.0)
        aggs = jnp.concatenate([mean, mn, mx, std], axis=1)
        amp_x = _xf(amp, 4 * f)
        att_x = _xf(att, 4 * f)
        aggs12 = jnp.concatenate([aggs, aggs * amp_x, aggs * att_x], axis=1)
        y = jnp.dot(aggs12, w_ref[...], preferred_element_type=jnp.float32)
        y = y + b_ref[0:1, :]
        y = jnp.maximum(y, 0.0)
        y_ref[...] = y

        ym = jnp.where(_valid(i, _BLK, h), y, 0.0)
        ps = jnp.sum(ym, axis=0, keepdims=True)
        pq = jnp.sum(ym * ym, axis=0, keepdims=True)

        @pl.when(i == 0)
        def _():
            st_ref[...] = jnp.zeros((8, h), jnp.float32)
        st_ref[0:1, :] += ps
        st_ref[1:2, :] += pq

    def run(s, q, mx, mn, degb, delta, w, bvec):
        return pl.pallas_call(
            body,
            grid=(NP80 // _BLK,),
            in_specs=[
                pl.BlockSpec((_BLK, f), lambda i: (i, 0)),
                pl.BlockSpec((_BLK, f), lambda i: (i, 0)),
                pl.BlockSpec((_BLK, f), lambda i: (i, 0)),
                pl.BlockSpec((_BLK, f), lambda i: (i, 0)),
                pl.BlockSpec((_BLK, 128), lambda i: (i, 0)),
                pl.BlockSpec((8, 128), lambda i: (0, 0)),
                pl.BlockSpec((12 * f, h), lambda i: (0, 0)),
                pl.BlockSpec((8, h), lambda i: (0, 0)),
            ],
            out_specs=[
                pl.BlockSpec((_BLK, h), lambda i: (i, 0)),
                pl.BlockSpec((8, h), lambda i: (0, 0)),
            ],
            out_shape=[
                jax.ShapeDtypeStruct((NP80, h), jnp.float32),
                jax.ShapeDtypeStruct((8, h), jnp.float32),
            ],
        )(s, q, mx, mn, degb, delta, w, bvec)

    return run


def _make_bn(h, final):
    def body(*refs):
        if final:
            (y_ref, st_ref, g_ref, be_ref, wc_ref, bc_ref, o_ref) = refs
        else:
            (y_ref, st_ref, g_ref, be_ref, o_ref) = refs
        mu = st_ref[0:1, :] / N
        var = st_ref[1:2, :] / N - mu * mu
        rsig = lax.rsqrt(var + 1e-5)
        hh = g_ref[0:1, :] * (y_ref[...] - mu) * rsig + be_ref[0:1, :]
        if final:
            o_ref[...] = jnp.dot(hh, wc_ref[...],
                                 preferred_element_type=jnp.float32) + bc_ref[0:1, :]
        else:
            o_ref[...] = hh

    def run(y, st, gv, bev, wc=None, bc=None):
        hout = 128 if final else h
        ins = [y, st, gv, bev] + ([wc, bc] if final else [])
        in_specs = [
            pl.BlockSpec((_BLK, h), lambda i: (i, 0)),
            pl.BlockSpec((8, h), lambda i: (0, 0)),
            pl.BlockSpec((8, h), lambda i: (0, 0)),
            pl.BlockSpec((8, h), lambda i: (0, 0)),
        ] + ([pl.BlockSpec((128, 128), lambda i: (0, 0)),
              pl.BlockSpec((8, 128), lambda i: (0, 0))] if final else [])
        return pl.pallas_call(
            body,
            grid=(NP80 // _BLK,),
            in_specs=in_specs,
            out_specs=pl.BlockSpec((_BLK, hout), lambda i: (i, 0)),
            out_shape=jax.ShapeDtypeStruct((NP80, hout), jnp.float32),
        )(*ins)

    return run


def _b8(v, h):
    return jnp.broadcast_to(v.reshape(1, -1), (8, h)).astype(jnp.float32)


def kernel(x, edge_index, W1, b1, g1, be1, W2, b2, g2, be2, W3, b3, g3, be3,
           W4, b4, g4, be4, Wc, bc):
    src = edge_index[0]
    dst = edge_index[1]

    counts = _make_hist()(dst)
    s_arr, meta = _scan(counts)
    ssrc, sdloc = _make_scatter()(src, dst, s_arr, meta)
    degb = _make_deg()(sdloc, meta)
    delta = _prep(degb)

    x80 = jnp.pad(x, ((0, NPAD - N), (0, 0))).reshape(NOWN, RPO, 128)
    x80 = jnp.pad(x80, ((0, 0), (0, 1), (0, 0))).reshape(NP80, 128)

    r1 = _make_reduce(NP80, 128, 2, 128)(x80, ssrc, sdloc, meta)
    y1, st1 = _make_layer(128, 128)(*r1, degb, delta, W1, _b8(b1, 128))
    h1 = _make_bn(128, False)(y1, st1, _b8(g1, 128), _b8(be1, 128))

    r2 = _make_reduce(NP80, 128, 2, 128)(h1, ssrc, sdloc, meta)
    y2, st2 = _make_layer(128, 256)(*r2, degb, delta, W2, _b8(b2, 256))
    h2 = _make_bn(256, False)(y2, st2, _b8(g2, 256), _b8(be2, 256))

    r3 = _make_reduce(NP80, 256, 1, 64)(h2, ssrc, sdloc, meta)
    y3, st3 = _make_layer(256, 128)(*r3, degb, delta, W3, _b8(b3, 128))
    h3 = _make_bn(128, False)(y3, st3, _b8(g3, 128), _b8(be3, 128))

    w4p = jnp.pad(W4, ((0, 0), (0, 64)))
    b4p = jnp.pad(b4, (0, 64))
    g4p = jnp.pad(g4, (0, 64))
    be4p = jnp.pad(be4, (0, 64))
    wcp = jnp.pad(Wc, ((0, 64), (0, 112)))
    bcp = jnp.pad(bc, (0, 112))
    r4 = _make_reduce(NP80, 128, 2, 128)(h3, ssrc, sdloc, meta)
    y4, st4 = _make_layer(128, 128)(*r4, degb, delta, w4p, _b8(b4p, 128))
    out = _make_bn(128, True)(y4, st4, _b8(g4p, 128), _b8(be4p, 128),
                              wcp, _b8(bcp, 128))
    out = out.reshape(NOWN, STRIDE, 128)[:, :RPO, :].reshape(NPAD, 128)
    return out[:N, :16]


# speedup vs baseline: 2.0198x; 1.0933x over previous
import functools

import jax
import jax.numpy as jnp
from jax import lax
from jax.experimental import pallas as pl
from jax.experimental.pallas import tpu as pltpu
from jax.experimental.pallas import tpu_sc as plsc

N = 10000
E = 320000
NOWN = 128
RPO = 79
STRIDE = 80
NW = 32
NPAD = NOWN * RPO
NP80 = NOWN * STRIDE
ECAP = E + NOWN * 128 + 128
NCH_E = E // 128
F32MAX = 3.4e38

_ioti = lambda: lax.iota(jnp.int32, 16)


def _div79(v):
    return lax.shift_right_logical(v * 53093, 22)


def _sx(vref, idx):
    return vref[pl.ds(idx, 16)][0]


def _lift16(scalars):
    v = jnp.zeros((16,), jnp.int32)
    it = _ioti()
    for l, s in enumerate(scalars):
        v = jnp.where(it == l, s, v)
    return v


def _scmesh():
    return plsc.VectorSubcoreMesh(core_axis_name="c", subcore_axis_name="s")


def _wid():
    return lax.axis_index("s") * 2 + lax.axis_index("c")


def _make_hist():
    @functools.partial(
        pl.kernel, mesh=_scmesh(),
        out_type=jax.ShapeDtypeStruct((NW, NOWN), jnp.int32),
        scratch_types=[
            pltpu.VMEM((128,), jnp.int32),
            pltpu.VMEM((128,), jnp.int32),
            pltpu.SMEM((128,), jnp.int32),
        ],
    )
    def hist(dst_hbm, counts_hbm, dbuf, cntv, cnt_s):
        w = _wid()

        def zi(i, c):
            cnt_s[i] = 0
            return c
        lax.fori_loop(0, 128, zi, 0)

        nch = jnp.where(w < NCH_E % NW, NCH_E // NW + 1, NCH_E // NW)

        def chunk(ci, c):
            base = pl.multiple_of((w + NW * ci) * 128, 128)
            pltpu.sync_copy(dst_hbm.at[pl.ds(base, 128)], dbuf.at[pl.ds(0, 128)])

            def kb(k, c2):
                dv = dbuf[pl.ds(pl.multiple_of(16 * k, 16), 16)]
                for l in range(16):
                    o = lax.shift_right_logical(dv[l] * 53093, 22)
                    cnt_s[o] = cnt_s[o] + 1
                return c2
            return lax.fori_loop(0, 8, kb, c)
        lax.fori_loop(0, nch, chunk, 0)

        def wv(j, c):
            return c
        for j in range(8):
            cntv[pl.ds(16 * j, 16)] = _lift16([cnt_s[16 * j + l] for l in range(16)])
        pltpu.sync_copy(cntv, counts_hbm.at[w])

    return hist


def _scan_body(cnt_ref, s_ref, meta_ref):
    cnt = cnt_ref[...]
    tot = jnp.sum(cnt, axis=0, keepdims=True)
    pt = ((tot + 127) // 128) * 128
    cpt = pt
    for k in (1, 2, 4, 8, 16, 32, 64):
        cpt = cpt + jnp.concatenate(
            [jnp.zeros((1, k), jnp.int32), cpt[:, :-k]], axis=1)
    ob = cpt - pt
    sumpt = jnp.max(cpt, axis=1, keepdims=True)
    cw = cnt
    for k in (1, 2, 4, 8, 16):
        cw = cw + jnp.concatenate(
            [jnp.zeros((k, NOWN), jnp.int32), cw[:-k, :]], axis=0)
    s_ref[...] = ob + cw - cnt
    meta = jnp.concatenate(
        [ob, tot, pt, jnp.broadcast_to(sumpt, (1, NOWN)),
         jnp.zeros((4, NOWN), jnp.int32)], axis=0)
    meta_ref[...] = meta


def _scan(counts):
    return pl.pallas_call(
        _scan_body,
        out_shape=[
            jax.ShapeDtypeStruct((NW, NOWN), jnp.int32),
            jax.ShapeDtypeStruct((8, NOWN), jnp.int32),
        ],
    )(counts)


def _make_scatter():
    @functools.partial(
        pl.kernel, mesh=_scmesh(),
        out_type=[
            jax.ShapeDtypeStruct((ECAP,), jnp.int32),
            jax.ShapeDtypeStruct((ECAP,), jnp.int32),
        ],
        scratch_types=[
            pltpu.VMEM((128,), jnp.int32),
            pltpu.VMEM((144,), jnp.int32),
            pltpu.VMEM((128,), jnp.int32),
            pltpu.VMEM((128,), jnp.int32),
            pltpu.VMEM((128,), jnp.int32),
            pltpu.VMEM((144,), jnp.int32),
            pltpu.VMEM((144,), jnp.int32),
            pltpu.VMEM((144,), jnp.int32),
            pltpu.VMEM((144,), jnp.int32),
            pltpu.VMEM((128,), jnp.int32),
            pltpu.VMEM((128,), jnp.int32),
            pltpu.SMEM((128,), jnp.int32),
            pltpu.SemaphoreType.DMA,
            pltpu.SemaphoreType.DMA,
        ],
    )
    def scat(src_hbm, dst_hbm, s_hbm, meta_hbm, ssrc_hbm, sdloc_hbm,
             sbuf, dbuf, posv, dlocv, srow, obv, totv, ptv, spv, zb, nb,
             start_s, sem1, sem2):
        w = _wid()
        it = _ioti()
        pltpu.sync_copy(s_hbm.at[w], srow)
        pltpu.sync_copy(meta_hbm.at[0], obv.at[pl.ds(0, 128)])
        pltpu.sync_copy(meta_hbm.at[1], totv.at[pl.ds(0, 128)])
        pltpu.sync_copy(meta_hbm.at[2], ptv.at[pl.ds(0, 128)])
        pltpu.sync_copy(meta_hbm.at[3], spv.at[pl.ds(0, 128)])
        for j in range(8):
            sv = srow[pl.ds(16 * j, 16)]
            for l in range(16):
                start_s[16 * j + l] = sv[l]
        z16 = jnp.zeros((16,), jnp.int32)
        n16 = jnp.full((16,), RPO, jnp.int32)
        for j in range(8):
            zb[pl.ds(16 * j, 16)] = z16
            nb[pl.ds(16 * j, 16)] = n16

        dump = jnp.int32(ECAP - NW) + w

        nch = jnp.where(w < NCH_E % NW, NCH_E // NW + 1, NCH_E // NW)

        def chunk(ci, c):
            base = pl.multiple_of((w + NW * ci) * 128, 128)
            pltpu.sync_copy(src_hbm.at[pl.ds(base, 128)], sbuf)
            pltpu.sync_copy(dst_hbm.at[pl.ds(base, 128)], dbuf.at[pl.ds(0, 128)])

            def kb(k, c2):
                sl = pl.ds(pl.multiple_of(16 * k, 16), 16)
                sv = sbuf[sl]
                sbuf[sl] = sv + _div79(sv)
                dv = dbuf[sl]
                dlocv[sl] = dv - _div79(dv) * RPO

                def lb(l, pos):
                    o = lax.shift_right_logical(
                        dbuf[pl.ds(16 * k + l, 16)][0] * 53093, 22)
                    p = start_s[o]
                    start_s[o] = p + 1
                    return jnp.where(_ioti() == l, p, pos)
                posv[sl] = lax.fori_loop(0, 16, lb, jnp.zeros((16,), jnp.int32))
                return c2
            lax.fori_loop(0, 8, kb, c)
            pltpu.async_copy(sbuf, ssrc_hbm.at[posv], sem1).wait()
            pltpu.async_copy(dlocv, sdloc_hbm.at[posv], sem2).wait()
            return c
        lax.fori_loop(0, nch, chunk, 0)

        for og in range(4):
            o = 4 * w + og
            ob_o = _sx(obv, o)
            tot_o = _sx(totv, o)
            pt_o = _sx(ptv, o)
            lim = ob_o + pt_o
            for j in range(8):
                p = ob_o + tot_o + it + 16 * j
                posv[pl.ds(16 * j, 16)] = jnp.where(p < lim, p, dump)
            pltpu.async_copy(zb, ssrc_hbm.at[posv], sem1).wait()
            pltpu.async_copy(nb, sdloc_hbm.at[posv], sem2).wait()

        sumpt = _sx(spv, jnp.int32(0))
        tail_lim = jnp.int32(ECAP - NW)

        def tl(t, c):
            base = sumpt + 128 * (w + NW * t)
            for j in range(8):
                p = base + it + 16 * j
                posv[pl.ds(16 * j, 16)] = jnp.where(p < tail_lim, p, dump)
            pltpu.async_copy(zb, ssrc_hbm.at[posv], sem1).wait()
            pltpu.async_copy(nb, sdloc_hbm.at[posv], sem2).wait()
            return c
        lax.fori_loop(0, 5, tl, 0)

        for j in range(8):
            posv[pl.ds(16 * j, 16)] = jnp.full((16,), dump, jnp.int32)
        pltpu.async_copy(zb, ssrc_hbm.at[posv], sem1).wait()
        pltpu.async_copy(nb, sdloc_hbm.at[posv], sem2).wait()

    return scat


def _make_deg():
    @functools.partial(
        pl.kernel, mesh=_scmesh(),
        out_type=jax.ShapeDtypeStruct((NP80, 128), jnp.float32),
        scratch_types=[
            pltpu.VMEM((128,), jnp.int32),
            pltpu.VMEM((STRIDE, 128), jnp.float32),
            pltpu.VMEM((144,), jnp.int32),
            pltpu.VMEM((144,), jnp.int32),
            pltpu.SMEM((STRIDE,), jnp.float32),
        ],
    )
    def deg(sdloc_hbm, meta_hbm, degb_hbm, dbuf, rowbuf, obv, ptv, cnt_s):
        w = _wid()
        pltpu.sync_copy(meta_hbm.at[0], obv.at[pl.ds(0, 128)])
        pltpu.sync_copy(meta_hbm.at[2], ptv.at[pl.ds(0, 128)])
        for og in range(4):
            o = 4 * w + og
            ob_o = _sx(obv, o)
            pt_o = _sx(ptv, o)

            def zi(i, c):
                cnt_s[i] = 0.0
                return c
            lax.fori_loop(0, STRIDE, zi, 0)

            def chunk(ci, c):
                pltpu.sync_copy(sdloc_hbm.at[pl.ds(pl.multiple_of(ob_o + 128 * ci, 128), 128)], dbuf)

                def kb(k, c2):
                    dv = dbuf[pl.ds(pl.multiple_of(16 * k, 16), 16)]
                    for l in range(16):
                        d = dv[l]
                        cnt_s[d] = cnt_s[d] + 1.0
                    return c2
                return lax.fori_loop(0, 8, kb, c)
            lax.fori_loop(0, pt_o // 128, chunk, 0)

            def wrow(r, c):
                v = jnp.full((16,), cnt_s[r], jnp.float32)
                for f in range(8):
                    rowbuf[r, pl.ds(16 * f, 16)] = v
                return c
            lax.fori_loop(0, STRIDE, wrow, 0)
            pltpu.sync_copy(rowbuf,
                            degb_hbm.at[pl.ds(pl.multiple_of(STRIDE * o, 8), STRIDE)])

    return deg


_BLK = 640


def _valid(i, b, h):
    g = lax.broadcasted_iota(jnp.int32, (b, h), 0) + i * b
    own = g // STRIDE
    rloc = g - own * STRIDE
    node = own * RPO + rloc
    return (rloc < RPO) & (node < N)


def _prep_body(degb_ref, delta_ref):
    i = pl.program_id(0)
    d = degb_ref[...]
    val = jnp.where(_valid(i, _BLK, 128), jnp.log(d + 1.0), 0.0)
    ps = jnp.sum(val, axis=0, keepdims=True) / N

    @pl.when(i == 0)
    def _():
        delta_ref[...] = jnp.zeros((8, 128), jnp.float32)
    delta_ref[0:1, :] += ps


def _prep(degb):
    return pl.pallas_call(
        _prep_body,
        grid=(NP80 // _BLK,),
        in_specs=[pl.BlockSpec((_BLK, 128), lambda i: (i, 0))],
        out_specs=pl.BlockSpec((8, 128), lambda i: (0, 0)),
        out_shape=jax.ShapeDtypeStruct((8, 128), jnp.float32),
    )(degb)


def _make_reduce(tn, f, opa, ch):
    rows_acc = STRIDE * opa
    rounds = 4 // opa
    nf = f // 16

    @functools.partial(
        pl.kernel, mesh=_scmesh(),
        out_type=[jax.ShapeDtypeStruct((NP80, f), jnp.float32)
                  for _ in range(4)],
        scratch_types=[
            pltpu.VMEM((ch,), jnp.int32),
            pltpu.VMEM((ch,), jnp.int32),
            pltpu.VMEM((ch + 16,), jnp.int32),
            pltpu.VMEM((ch + 16,), jnp.int32),
            pltpu.VMEM((ch, f), jnp.float32),
            pltpu.VMEM((ch, f), jnp.float32),
            pltpu.VMEM((rows_acc, f), jnp.float32),
            pltpu.VMEM((rows_acc, f), jnp.float32),
            pltpu.VMEM((rows_acc, f), jnp.float32),
            pltpu.VMEM((rows_acc, f), jnp.float32),
            pltpu.VMEM((144,), jnp.int32),
            pltpu.VMEM((144,), jnp.int32),
            pltpu.SemaphoreType.DMA,
            pltpu.SemaphoreType.DMA,
            pltpu.SemaphoreType.DMA,
            pltpu.SemaphoreType.DMA,
        ],
    )
    def red(table_hbm, ssrc_hbm, sdloc_hbm, meta_hbm,
            os_hbm, oq_hbm, om_hbm, on_hbm,
            sidx_a, sidx_b, dl_a, dl_b, rows_a, rows_b,
            acc_s, acc_q, acc_m, acc_n, obv, ptv,
            isem_a, isem_b, gsem_a, gsem_b):
        w = _wid()
        pltpu.sync_copy(meta_hbm.at[0], obv.at[pl.ds(0, 128)])
        pltpu.sync_copy(meta_hbm.at[2], ptv.at[pl.ds(0, 128)])
        z16 = jnp.zeros((16,), jnp.float32)
        lo16 = jnp.full((16,), -F32MAX, jnp.float32)
        hi16 = jnp.full((16,), F32MAX, jnp.float32)
        slots = ((sidx_a, dl_a, rows_a, isem_a, gsem_a),
                 (sidx_b, dl_b, rows_b, isem_b, gsem_b))

        for rnd in range(rounds):
            def zrow(r, c):
                for fi in range(nf):
                    sl = pl.ds(16 * fi, 16)
                    acc_s[r, sl] = z16
                    acc_q[r, sl] = z16
                    acc_m[r, sl] = lo16
                    acc_n[r, sl] = hi16
                return c
            lax.fori_loop(0, rows_acc, zrow, 0)

            for seg in range(opa):
                o = 4 * w + rnd * opa + seg
                ob_o = _sx(obv, o)
                pt_o = _sx(ptv, o)
                rbase = STRIDE * seg
                nch = pt_o // ch

                def issue_idx(c, s):
                    base = pl.multiple_of(ob_o + ch * c, ch)
                    pltpu.async_copy(ssrc_hbm.at[pl.ds(base, ch)], s[0], s[3])
                    pltpu.async_copy(sdloc_hbm.at[pl.ds(base, ch)],
                                     s[1].at[pl.ds(0, ch)], s[3])

                def wait_idx(s):
                    pltpu.make_async_copy(ssrc_hbm.at[pl.ds(0, ch)], s[0], s[3]).wait()
                    pltpu.make_async_copy(ssrc_hbm.at[pl.ds(0, ch)],
                                          s[1].at[pl.ds(0, ch)], s[3]).wait()

                def issue_gather(s):
                    pltpu.async_copy(table_hbm.at[s[0]], s[2], s[4])

                def wait_gather(s):
                    pltpu.make_async_copy(table_hbm.at[pl.ds(0, ch)], s[2], s[4]).wait()

                def process(s):
                    dlb, rws = s[1], s[2]

                    def eb(e, c2):
                        row = rbase + dlb[pl.ds(e, 16)][0]
                        for fi in range(nf):
                            sl = pl.ds(16 * fi, 16)
                            r = rws[e, sl]
                            acc_s[row, sl] += r
                            acc_q[row, sl] += r * r
                            acc_m[row, sl] = jnp.maximum(acc_m[row, sl], r)
                            acc_n[row, sl] = jnp.minimum(acc_n[row, sl], r)
                        return c2
                    lax.fori_loop(0, ch, eb, 0)

                @pl.when(nch > 0)
                def _():
                    issue_idx(0, slots[0])
                    wait_idx(slots[0])
                    issue_gather(slots[0])

                @pl.when(nch > 1)
                def _():
                    issue_idx(1, slots[1])

                def pair(t, c):
                    for b in range(2):
                        ci = 2 * t + b
                        s_cur = slots[b]
                        s_oth = slots[1 - b]

                        @pl.when(ci < nch)
                        def _():
                            @pl.when(ci + 1 < nch)
                            def _():
                                wait_idx(s_oth)
                                issue_gather(s_oth)
                            wait_gather(s_cur)
                            process(s_cur)

                            @pl.when(ci + 2 < nch)
                            def _():
                                issue_idx(ci + 2, s_cur)
                    return c
                lax.fori_loop(0, (nch + 1) // 2, pair, 0)

                dst_rows = pl.ds(pl.multiple_of(STRIDE * o, 8), STRIDE)
                src_rows = pl.ds(rbase, STRIDE)
                pltpu.sync_copy(acc_s.at[src_rows], os_hbm.at[dst_rows])
                pltpu.sync_copy(acc_q.at[src_rows], oq_hbm.at[dst_rows])
                pltpu.sync_copy(acc_m.at[src_rows], om_hbm.at[dst_rows])
                pltpu.sync_copy(acc_n.at[src_rows], on_hbm.at[dst_rows])

    return red


def _xf(v, f):
    if f == 128:
        return v
    return jnp.concatenate([v] * (f // 128), axis=1)


def _make_layer(f, h):
    def body(s_ref, q_ref, mx_ref, mn_ref, degb_ref, delta_ref, w_ref, b_ref,
             y_ref, st_ref):
        i = pl.program_id(0)
        deg = degb_ref[...]
        dl = delta_ref[0:1, :]
        degc = jnp.maximum(deg, 1.0)
        logd = jnp.log(deg + 1.0)
        amp = logd / dl
        att = dl / jnp.maximum(logd, 1e-5)
        has = deg > 0.0

        degc_x = _xf(degc, f)
        has_x = _xf(has, f)
        s = s_ref[...]
        q = q_ref[...]
        mean = s / degc_x
        std = jnp.sqrt(jnp.maximum(q / degc_x - mean * mean, 0.0) + 1e-5)
        mx = jnp.where(has_x, mx_ref[...], 0.0)
        mn = jnp.where(has_x, mn_ref[...], 0.0)
        aggs = jnp.concatenate([mean, mn, mx, std], axis=1)
        amp_x = _xf(amp, 4 * f)
        att_x = _xf(att, 4 * f)
        aggs12 = jnp.concatenate([aggs, aggs * amp_x, aggs * att_x], axis=1)
        y = jnp.dot(aggs12, w_ref[...], preferred_element_type=jnp.float32)
        y = y + b_ref[0:1, :]
        y = jnp.maximum(y, 0.0)
        y_ref[...] = y

        ym = jnp.where(_valid(i, _BLK, h), y, 0.0)
        ps = jnp.sum(ym, axis=0, keepdims=True)
        pq = jnp.sum(ym * ym, axis=0, keepdims=True)

        @pl.when(i == 0)
        def _():
            st_ref[...] = jnp.zeros((8, h), jnp.float32)
        st_ref[0:1, :] += ps
        st_ref[1:2, :] += pq

    def run(s, q, mx, mn, degb, delta, w, bvec):
        return pl.pallas_call(
            body,
            grid=(NP80 // _BLK,),
            in_specs=[
                pl.BlockSpec((_BLK, f), lambda i: (i, 0)),
                pl.BlockSpec((_BLK, f), lambda i: (i, 0)),
                pl.BlockSpec((_BLK, f), lambda i: (i, 0)),
                pl.BlockSpec((_BLK, f), lambda i: (i, 0)),
                pl.BlockSpec((_BLK, 128), lambda i: (i, 0)),
                pl.BlockSpec((8, 128), lambda i: (0, 0)),
                pl.BlockSpec((12 * f, h), lambda i: (0, 0)),
                pl.BlockSpec((8, h), lambda i: (0, 0)),
            ],
            out_specs=[
                pl.BlockSpec((_BLK, h), lambda i: (i, 0)),
                pl.BlockSpec((8, h), lambda i: (0, 0)),
            ],
            out_shape=[
                jax.ShapeDtypeStruct((NP80, h), jnp.float32),
                jax.ShapeDtypeStruct((8, h), jnp.float32),
            ],
        )(s, q, mx, mn, degb, delta, w, bvec)

    return run


def _make_bn(h, final):
    def body(*refs):
        if final:
            (y_ref, st_ref, g_ref, be_ref, wc_ref, bc_ref, o_ref) = refs
        else:
            (y_ref, st_ref, g_ref, be_ref, o_ref) = refs
        mu = st_ref[0:1, :] / N
        var = st_ref[1:2, :] / N - mu * mu
        rsig = lax.rsqrt(var + 1e-5)
        hh = g_ref[0:1, :] * (y_ref[...] - mu) * rsig + be_ref[0:1, :]
        if final:
            o_ref[...] = jnp.dot(hh, wc_ref[...],
                                 preferred_element_type=jnp.float32) + bc_ref[0:1, :]
        else:
            o_ref[...] = hh

    def run(y, st, gv, bev, wc=None, bc=None):
        hout = 128 if final else h
        ins = [y, st, gv, bev] + ([wc, bc] if final else [])
        in_specs = [
            pl.BlockSpec((_BLK, h), lambda i: (i, 0)),
            pl.BlockSpec((8, h), lambda i: (0, 0)),
            pl.BlockSpec((8, h), lambda i: (0, 0)),
            pl.BlockSpec((8, h), lambda i: (0, 0)),
        ] + ([pl.BlockSpec((128, 128), lambda i: (0, 0)),
              pl.BlockSpec((8, 128), lambda i: (0, 0))] if final else [])
        return pl.pallas_call(
            body,
            grid=(NP80 // _BLK,),
            in_specs=in_specs,
            out_specs=pl.BlockSpec((_BLK, hout), lambda i: (i, 0)),
            out_shape=jax.ShapeDtypeStruct((NP80, hout), jnp.float32),
        )(*ins)

    return run


def _b8(v, h):
    return jnp.broadcast_to(v.reshape(1, -1), (8, h)).astype(jnp.float32)


def kernel(x, edge_index, W1, b1, g1, be1, W2, b2, g2, be2, W3, b3, g3, be3,
           W4, b4, g4, be4, Wc, bc):
    src = edge_index[0]
    dst = edge_index[1]

    counts = _make_hist()(dst)
    s_arr, meta = _scan(counts)
    ssrc, sdloc = _make_scatter()(src, dst, s_arr, meta)
    degb = _make_deg()(sdloc, meta)
    delta = _prep(degb)

    x80 = jnp.pad(x, ((0, NPAD - N), (0, 0))).reshape(NOWN, RPO, 128)
    x80 = jnp.pad(x80, ((0, 0), (0, 1), (0, 0))).reshape(NP80, 128)

    r1 = _make_reduce(NP80, 128, 2, 128)(x80, ssrc, sdloc, meta)
    y1, st1 = _make_layer(128, 128)(*r1, degb, delta, W1, _b8(b1, 128))
    h1 = _make_bn(128, False)(y1, st1, _b8(g1, 128), _b8(be1, 128))

    r2 = _make_reduce(NP80, 128, 2, 128)(h1, ssrc, sdloc, meta)
    y2, st2 = _make_layer(128, 256)(*r2, degb, delta, W2, _b8(b2, 256))
    h2 = _make_bn(256, False)(y2, st2, _b8(g2, 256), _b8(be2, 256))

    r3 = _make_reduce(NP80, 256, 1, 64)(h2, ssrc, sdloc, meta)
    y3, st3 = _make_layer(256, 128)(*r3, degb, delta, W3, _b8(b3, 128))
    h3 = _make_bn(128, False)(y3, st3, _b8(g3, 128), _b8(be3, 128))

    w4p = jnp.pad(W4, ((0, 0), (0, 64)))
    b4p = jnp.pad(b4, (0, 64))
    g4p = jnp.pad(g4, (0, 64))
    be4p = jnp.pad(be4, (0, 64))
    wcp = jnp.pad(Wc, ((0, 64), (0, 112)))
    bcp = jnp.pad(bc, (0, 112))
    r4 = _make_reduce(NP80, 128, 2, 128)(h3, ssrc, sdloc, meta)
    y4, st4 = _make_layer(128, 128)(*r4, degb, delta, w4p, _b8(b4p, 128))
    out = _make_bn(128, True)(y4, st4, _b8(g4p, 128), _b8(be4p, 128),
                              wcp, _b8(bcp, 128))
    out = out.reshape(NOWN, STRIDE, 128)[:, :RPO, :].reshape(NPAD, 128)
    return out[:N, :16]


# speedup vs baseline: 2.0200x; 1.0001x over previous
import functools

import jax
import jax.numpy as jnp
from jax import lax
from jax.experimental import pallas as pl
from jax.experimental.pallas import tpu as pltpu
from jax.experimental.pallas import tpu_sc as plsc

N = 10000
E = 320000
NOWN = 128
RPO = 79
STRIDE = 80
NW = 32
NPAD = NOWN * RPO
NP80 = NOWN * STRIDE
ECAP = E + NOWN * 128 + 128
NCH_E = E // 128
F32MAX = 3.4e38

_ioti = lambda: lax.iota(jnp.int32, 16)


def _div79(v):
    return lax.shift_right_logical(v * 53093, 22)


def _sx(vref, idx):
    return vref[pl.ds(idx, 16)][0]


def _lift16(scalars):
    v = jnp.zeros((16,), jnp.int32)
    it = _ioti()
    for l, s in enumerate(scalars):
        v = jnp.where(it == l, s, v)
    return v


def _scmesh():
    return plsc.VectorSubcoreMesh(core_axis_name="c", subcore_axis_name="s")


def _wid():
    return lax.axis_index("s") * 2 + lax.axis_index("c")


def _make_hist():
    @functools.partial(
        pl.kernel, mesh=_scmesh(),
        out_type=jax.ShapeDtypeStruct((NW, NOWN), jnp.int32),
        scratch_types=[
            pltpu.VMEM((128,), jnp.int32),
            pltpu.VMEM((128,), jnp.int32),
            pltpu.SMEM((128,), jnp.int32),
        ],
    )
    def hist(dst_hbm, counts_hbm, dbuf, cntv, cnt_s):
        w = _wid()

        def zi(i, c):
            cnt_s[i] = 0
            return c
        lax.fori_loop(0, 128, zi, 0)

        nch = jnp.where(w < NCH_E % NW, NCH_E // NW + 1, NCH_E // NW)

        def chunk(ci, c):
            base = pl.multiple_of((w + NW * ci) * 128, 128)
            pltpu.sync_copy(dst_hbm.at[pl.ds(base, 128)], dbuf.at[pl.ds(0, 128)])

            def kb(k, c2):
                dv = dbuf[pl.ds(pl.multiple_of(16 * k, 16), 16)]
                for l in range(16):
                    o = lax.shift_right_logical(dv[l] * 53093, 22)
                    cnt_s[o] = cnt_s[o] + 1
                return c2
            return lax.fori_loop(0, 8, kb, c)
        lax.fori_loop(0, nch, chunk, 0)

        def wv(j, c):
            return c
        for j in range(8):
            cntv[pl.ds(16 * j, 16)] = _lift16([cnt_s[16 * j + l] for l in range(16)])
        pltpu.sync_copy(cntv, counts_hbm.at[w])

    return hist


def _scan_body(cnt_ref, s_ref, meta_ref):
    cnt = cnt_ref[...]
    tot = jnp.sum(cnt, axis=0, keepdims=True)
    pt = ((tot + 127) // 128) * 128
    cpt = pt
    for k in (1, 2, 4, 8, 16, 32, 64):
        cpt = cpt + jnp.concatenate(
            [jnp.zeros((1, k), jnp.int32), cpt[:, :-k]], axis=1)
    ob = cpt - pt
    sumpt = jnp.max(cpt, axis=1, keepdims=True)
    cw = cnt
    for k in (1, 2, 4, 8, 16):
        cw = cw + jnp.concatenate(
            [jnp.zeros((k, NOWN), jnp.int32), cw[:-k, :]], axis=0)
    s_ref[...] = ob + cw - cnt
    meta = jnp.concatenate(
        [ob, tot, pt, jnp.broadcast_to(sumpt, (1, NOWN)),
         jnp.zeros((4, NOWN), jnp.int32)], axis=0)
    meta_ref[...] = meta


def _scan(counts):
    return pl.pallas_call(
        _scan_body,
        out_shape=[
            jax.ShapeDtypeStruct((NW, NOWN), jnp.int32),
            jax.ShapeDtypeStruct((8, NOWN), jnp.int32),
        ],
    )(counts)


def _make_scatter():
    @functools.partial(
        pl.kernel, mesh=_scmesh(),
        out_type=[
            jax.ShapeDtypeStruct((ECAP,), jnp.int32),
            jax.ShapeDtypeStruct((ECAP,), jnp.int32),
        ],
        scratch_types=[
            pltpu.VMEM((128,), jnp.int32),
            pltpu.VMEM((144,), jnp.int32),
            pltpu.VMEM((128,), jnp.int32),
            pltpu.VMEM((128,), jnp.int32),
            pltpu.VMEM((128,), jnp.int32),
            pltpu.VMEM((144,), jnp.int32),
            pltpu.VMEM((144,), jnp.int32),
            pltpu.VMEM((144,), jnp.int32),
            pltpu.VMEM((144,), jnp.int32),
            pltpu.VMEM((128,), jnp.int32),
            pltpu.VMEM((128,), jnp.int32),
            pltpu.SMEM((128,), jnp.int32),
            pltpu.SemaphoreType.DMA,
            pltpu.SemaphoreType.DMA,
        ],
    )
    def scat(src_hbm, dst_hbm, s_hbm, meta_hbm, ssrc_hbm, sdloc_hbm,
             sbuf, dbuf, posv, dlocv, srow, obv, totv, ptv, spv, zb, nb,
             start_s, sem1, sem2):
        w = _wid()
        it = _ioti()
        pltpu.sync_copy(s_hbm.at[w], srow)
        pltpu.sync_copy(meta_hbm.at[0], obv.at[pl.ds(0, 128)])
        pltpu.sync_copy(meta_hbm.at[1], totv.at[pl.ds(0, 128)])
        pltpu.sync_copy(meta_hbm.at[2], ptv.at[pl.ds(0, 128)])
        pltpu.sync_copy(meta_hbm.at[3], spv.at[pl.ds(0, 128)])
        for j in range(8):
            sv = srow[pl.ds(16 * j, 16)]
            for l in range(16):
                start_s[16 * j + l] = sv[l]
        z16 = jnp.zeros((16,), jnp.int32)
        n16 = jnp.full((16,), RPO, jnp.int32)
        for j in range(8):
            zb[pl.ds(16 * j, 16)] = z16
            nb[pl.ds(16 * j, 16)] = n16

        dump = jnp.int32(ECAP - NW) + w

        nch = jnp.where(w < NCH_E % NW, NCH_E // NW + 1, NCH_E // NW)

        def chunk(ci, c):
            base = pl.multiple_of((w + NW * ci) * 128, 128)
            pltpu.sync_copy(src_hbm.at[pl.ds(base, 128)], sbuf)
            pltpu.sync_copy(dst_hbm.at[pl.ds(base, 128)], dbuf.at[pl.ds(0, 128)])

            def kb(k, c2):
                sl = pl.ds(pl.multiple_of(16 * k, 16), 16)
                sv = sbuf[sl]
                sbuf[sl] = sv + _div79(sv)
                dv = dbuf[sl]
                ov = _div79(dv)
                dlocv[sl] = dv - ov * RPO
                scal = []
                for l in range(16):
                    o = ov[l]
                    p = start_s[o]
                    start_s[o] = p + 1
                    scal.append(p)
                posv[sl] = _lift16(scal)
                return c2
            lax.fori_loop(0, 8, kb, c)
            pltpu.async_copy(sbuf, ssrc_hbm.at[posv], sem1).wait()
            pltpu.async_copy(dlocv, sdloc_hbm.at[posv], sem2).wait()
            return c
        lax.fori_loop(0, nch, chunk, 0)

        for og in range(4):
            o = 4 * w + og
            ob_o = _sx(obv, o)
            tot_o = _sx(totv, o)
            pt_o = _sx(ptv, o)
            lim = ob_o + pt_o
            for j in range(8):
                p = ob_o + tot_o + it + 16 * j
                posv[pl.ds(16 * j, 16)] = jnp.where(p < lim, p, dump)
            pltpu.async_copy(zb, ssrc_hbm.at[posv], sem1).wait()
            pltpu.async_copy(nb, sdloc_hbm.at[posv], sem2).wait()

        sumpt = _sx(spv, jnp.int32(0))
        tail_lim = jnp.int32(ECAP - NW)

        def tl(t, c):
            base = sumpt + 128 * (w + NW * t)
            for j in range(8):
                p = base + it + 16 * j
                posv[pl.ds(16 * j, 16)] = jnp.where(p < tail_lim, p, dump)
            pltpu.async_copy(zb, ssrc_hbm.at[posv], sem1).wait()
            pltpu.async_copy(nb, sdloc_hbm.at[posv], sem2).wait()
            return c
        lax.fori_loop(0, 5, tl, 0)

        for j in range(8):
            posv[pl.ds(16 * j, 16)] = jnp.full((16,), dump, jnp.int32)
        pltpu.async_copy(zb, ssrc_hbm.at[posv], sem1).wait()
        pltpu.async_copy(nb, sdloc_hbm.at[posv], sem2).wait()

    return scat


def _make_deg():
    @functools.partial(
        pl.kernel, mesh=_scmesh(),
        out_type=jax.ShapeDtypeStruct((NP80, 128), jnp.float32),
        scratch_types=[
            pltpu.VMEM((128,), jnp.int32),
            pltpu.VMEM((STRIDE, 128), jnp.float32),
            pltpu.VMEM((144,), jnp.int32),
            pltpu.VMEM((144,), jnp.int32),
            pltpu.SMEM((STRIDE,), jnp.float32),
        ],
    )
    def deg(sdloc_hbm, meta_hbm, degb_hbm, dbuf, rowbuf, obv, ptv, cnt_s):
        w = _wid()
        pltpu.sync_copy(meta_hbm.at[0], obv.at[pl.ds(0, 128)])
        pltpu.sync_copy(meta_hbm.at[2], ptv.at[pl.ds(0, 128)])
        for og in range(4):
            o = 4 * w + og
            ob_o = _sx(obv, o)
            pt_o = _sx(ptv, o)

            def zi(i, c):
                cnt_s[i] = 0.0
                return c
            lax.fori_loop(0, STRIDE, zi, 0)

            def chunk(ci, c):
                pltpu.sync_copy(sdloc_hbm.at[pl.ds(pl.multiple_of(ob_o + 128 * ci, 128), 128)], dbuf)

                def kb(k, c2):
                    dv = dbuf[pl.ds(pl.multiple_of(16 * k, 16), 16)]
                    for l in range(16):
                        d = dv[l]
                        cnt_s[d] = cnt_s[d] + 1.0
                    return c2
                return lax.fori_loop(0, 8, kb, c)
            lax.fori_loop(0, pt_o // 128, chunk, 0)

            def wrow(r, c):
                v = jnp.full((16,), cnt_s[r], jnp.float32)
                for f in range(8):
                    rowbuf[r, pl.ds(16 * f, 16)] = v
                return c
            lax.fori_loop(0, STRIDE, wrow, 0)
            pltpu.sync_copy(rowbuf,
                            degb_hbm.at[pl.ds(pl.multiple_of(STRIDE * o, 8), STRIDE)])

    return deg


_BLK = 640


def _valid(i, b, h):
    g = lax.broadcasted_iota(jnp.int32, (b, h), 0) + i * b
    own = g // STRIDE
    rloc = g - own * STRIDE
    node = own * RPO + rloc
    return (rloc < RPO) & (node < N)


def _prep_body(degb_ref, delta_ref):
    i = pl.program_id(0)
    d = degb_ref[...]
    val = jnp.where(_valid(i, _BLK, 128), jnp.log(d + 1.0), 0.0)
    ps = jnp.sum(val, axis=0, keepdims=True) / N

    @pl.when(i == 0)
    def _():
        delta_ref[...] = jnp.zeros((8, 128), jnp.float32)
    delta_ref[0:1, :] += ps


def _prep(degb):
    return pl.pallas_call(
        _prep_body,
        grid=(NP80 // _BLK,),
        in_specs=[pl.BlockSpec((_BLK, 128), lambda i: (i, 0))],
        out_specs=pl.BlockSpec((8, 128), lambda i: (0, 0)),
        out_shape=jax.ShapeDtypeStruct((8, 128), jnp.float32),
    )(degb)


def _make_reduce(tn, f, opa, ch):
    rows_acc = STRIDE * opa
    rounds = 4 // opa
    nf = f // 16

    @functools.partial(
        pl.kernel, mesh=_scmesh(),
        out_type=[jax.ShapeDtypeStruct((NP80, f), jnp.float32)
                  for _ in range(4)],
        scratch_types=[
            pltpu.VMEM((ch,), jnp.int32),
            pltpu.VMEM((ch,), jnp.int32),
            pltpu.VMEM((ch + 16,), jnp.int32),
            pltpu.VMEM((ch + 16,), jnp.int32),
            pltpu.VMEM((ch, f), jnp.float32),
            pltpu.VMEM((ch, f), jnp.float32),
            pltpu.VMEM((rows_acc, f), jnp.float32),
            pltpu.VMEM((rows_acc, f), jnp.float32),
            pltpu.VMEM((rows_acc, f), jnp.float32),
            pltpu.VMEM((rows_acc, f), jnp.float32),
            pltpu.VMEM((144,), jnp.int32),
            pltpu.VMEM((144,), jnp.int32),
            pltpu.SemaphoreType.DMA,
            pltpu.SemaphoreType.DMA,
            pltpu.SemaphoreType.DMA,
            pltpu.SemaphoreType.DMA,
        ],
    )
    def red(table_hbm, ssrc_hbm, sdloc_hbm, meta_hbm,
            os_hbm, oq_hbm, om_hbm, on_hbm,
            sidx_a, sidx_b, dl_a, dl_b, rows_a, rows_b,
            acc_s, acc_q, acc_m, acc_n, obv, ptv,
            isem_a, isem_b, gsem_a, gsem_b):
        w = _wid()
        pltpu.sync_copy(meta_hbm.at[0], obv.at[pl.ds(0, 128)])
        pltpu.sync_copy(meta_hbm.at[2], ptv.at[pl.ds(0, 128)])
        z16 = jnp.zeros((16,), jnp.float32)
        lo16 = jnp.full((16,), -F32MAX, jnp.float32)
        hi16 = jnp.full((16,), F32MAX, jnp.float32)
        slots = ((sidx_a, dl_a, rows_a, isem_a, gsem_a),
                 (sidx_b, dl_b, rows_b, isem_b, gsem_b))

        for rnd in range(rounds):
            def zrow(r, c):
                for fi in range(nf):
                    sl = pl.ds(16 * fi, 16)
                    acc_s[r, sl] = z16
                    acc_q[r, sl] = z16
                    acc_m[r, sl] = lo16
                    acc_n[r, sl] = hi16
                return c
            lax.fori_loop(0, rows_acc, zrow, 0)

            for seg in range(opa):
                o = 4 * w + rnd * opa + seg
                ob_o = _sx(obv, o)
                pt_o = _sx(ptv, o)
                rbase = STRIDE * seg
                nch = pt_o // ch

                def issue_idx(c, s):
                    base = pl.multiple_of(ob_o + ch * c, ch)
                    pltpu.async_copy(ssrc_hbm.at[pl.ds(base, ch)], s[0], s[3])
                    pltpu.async_copy(sdloc_hbm.at[pl.ds(base, ch)],
                                     s[1].at[pl.ds(0, ch)], s[3])

                def wait_idx(s):
                    pltpu.make_async_copy(ssrc_hbm.at[pl.ds(0, ch)], s[0], s[3]).wait()
                    pltpu.make_async_copy(ssrc_hbm.at[pl.ds(0, ch)],
                                          s[1].at[pl.ds(0, ch)], s[3]).wait()

                def issue_gather(s):
                    pltpu.async_copy(table_hbm.at[s[0]], s[2], s[4])

                def wait_gather(s):
                    pltpu.make_async_copy(table_hbm.at[pl.ds(0, ch)], s[2], s[4]).wait()

                def process(s):
                    dlb, rws = s[1], s[2]

                    def eb(e, c2):
                        row = rbase + dlb[pl.ds(e, 16)][0]
                        for fi in range(nf):
                            sl = pl.ds(16 * fi, 16)
                            r = rws[e, sl]
                            acc_s[row, sl] += r
                            acc_q[row, sl] += r * r
                            acc_m[row, sl] = jnp.maximum(acc_m[row, sl], r)
                            acc_n[row, sl] = jnp.minimum(acc_n[row, sl], r)
                        return c2
                    lax.fori_loop(0, ch, eb, 0)

                @pl.when(nch > 0)
                def _():
                    issue_idx(0, slots[0])
                    wait_idx(slots[0])
                    issue_gather(slots[0])

                @pl.when(nch > 1)
                def _():
                    issue_idx(1, slots[1])

                def pair(t, c):
                    for b in range(2):
                        ci = 2 * t + b
                        s_cur = slots[b]
                        s_oth = slots[1 - b]

                        @pl.when(ci < nch)
                        def _():
                            @pl.when(ci + 1 < nch)
                            def _():
                                wait_idx(s_oth)
                                issue_gather(s_oth)
                            wait_gather(s_cur)
                            process(s_cur)

                            @pl.when(ci + 2 < nch)
                            def _():
                                issue_idx(ci + 2, s_cur)
                    return c
                lax.fori_loop(0, (nch + 1) // 2, pair, 0)

                dst_rows = pl.ds(pl.multiple_of(STRIDE * o, 8), STRIDE)
                src_rows = pl.ds(rbase, STRIDE)
                pltpu.sync_copy(acc_s.at[src_rows], os_hbm.at[dst_rows])
                pltpu.sync_copy(acc_q.at[src_rows], oq_hbm.at[dst_rows])
                pltpu.sync_copy(acc_m.at[src_rows], om_hbm.at[dst_rows])
                pltpu.sync_copy(acc_n.at[src_rows], on_hbm.at[dst_rows])

    return red


def _xf(v, f):
    if f == 128:
        return v
    return jnp.concatenate([v] * (f // 128), axis=1)


def _make_layer(f, h):
    def body(s_ref, q_ref, mx_ref, mn_ref, degb_ref, delta_ref, w_ref, b_ref,
             y_ref, st_ref):
        i = pl.program_id(0)
        deg = degb_ref[...]
        dl = delta_ref[0:1, :]
        degc = jnp.maximum(deg, 1.0)
        logd = jnp.log(deg + 1.0)
        amp = logd / dl
        att = dl / jnp.maximum(logd, 1e-5)
        has = deg > 0.0

        degc_x = _xf(degc, f)
        has_x = _xf(has, f)
        s = s_ref[...]
        q = q_ref[...]
        mean = s / degc_x
        std = jnp.sqrt(jnp.maximum(q / degc_x - mean * mean, 0.0) + 1e-5)
        mx = jnp.where(has_x, mx_ref[...], 0.0)
        mn = jnp.where(has_x, mn_ref[...], 0.0)
        aggs = jnp.concatenate([mean, mn, mx, std], axis=1)
        amp_x = _xf(amp, 4 * f)
        att_x = _xf(att, 4 * f)
        aggs12 = jnp.concatenate([aggs, aggs * amp_x, aggs * att_x], axis=1)
        y = jnp.dot(aggs12, w_ref[...], preferred_element_type=jnp.float32)
        y = y + b_ref[0:1, :]
        y = jnp.maximum(y, 0.0)
        y_ref[...] = y

        ym = jnp.where(_valid(i, _BLK, h), y, 0.0)
        ps = jnp.sum(ym, axis=0, keepdims=True)
        pq = jnp.sum(ym * ym, axis=0, keepdims=True)

        @pl.when(i == 0)
        def _():
            st_ref[...] = jnp.zeros((8, h), jnp.float32)
        st_ref[0:1, :] += ps
        st_ref[1:2, :] += pq

    def run(s, q, mx, mn, degb, delta, w, bvec):
        return pl.pallas_call(
            body,
            grid=(NP80 // _BLK,),
            in_specs=[
                pl.BlockSpec((_BLK, f), lambda i: (i, 0)),
                pl.BlockSpec((_BLK, f), lambda i: (i, 0)),
                pl.BlockSpec((_BLK, f), lambda i: (i, 0)),
                pl.BlockSpec((_BLK, f), lambda i: (i, 0)),
                pl.BlockSpec((_BLK, 128), lambda i: (i, 0)),
                pl.BlockSpec((8, 128), lambda i: (0, 0)),
                pl.BlockSpec((12 * f, h), lambda i: (0, 0)),
                pl.BlockSpec((8, h), lambda i: (0, 0)),
            ],
            out_specs=[
                pl.BlockSpec((_BLK, h), lambda i: (i, 0)),
                pl.BlockSpec((8, h), lambda i: (0, 0)),
            ],
            out_shape=[
                jax.ShapeDtypeStruct((NP80, h), jnp.float32),
                jax.ShapeDtypeStruct((8, h), jnp.float32),
            ],
        )(s, q, mx, mn, degb, delta, w, bvec)

    return run


def _make_bn(h, final):
    def body(*refs):
        if final:
            (y_ref, st_ref, g_ref, be_ref, wc_ref, bc_ref, o_ref) = refs
        else:
            (y_ref, st_ref, g_ref, be_ref, o_ref) = refs
        mu = st_ref[0:1, :] / N
        var = st_ref[1:2, :] / N - mu * mu
        rsig = lax.rsqrt(var + 1e-5)
        hh = g_ref[0:1, :] * (y_ref[...] - mu) * rsig + be_ref[0:1, :]
        if final:
            o_ref[...] = jnp.dot(hh, wc_ref[...],
                                 preferred_element_type=jnp.float32) + bc_ref[0:1, :]
        else:
            o_ref[...] = hh

    def run(y, st, gv, bev, wc=None, bc=None):
        hout = 128 if final else h
        ins = [y, st, gv, bev] + ([wc, bc] if final else [])
        in_specs = [
            pl.BlockSpec((_BLK, h), lambda i: (i, 0)),
            pl.BlockSpec((8, h), lambda i: (0, 0)),
            pl.BlockSpec((8, h), lambda i: (0, 0)),
            pl.BlockSpec((8, h), lambda i: (0, 0)),
        ] + ([pl.BlockSpec((128, 128), lambda i: (0, 0)),
              pl.BlockSpec((8, 128), lambda i: (0, 0))] if final else [])
        return pl.pallas_call(
            body,
            grid=(NP80 // _BLK,),
            in_specs=in_specs,
            out_specs=pl.BlockSpec((_BLK, hout), lambda i: (i, 0)),
            out_shape=jax.ShapeDtypeStruct((NP80, hout), jnp.float32),
        )(*ins)

    return run


def _b8(v, h):
    return jnp.broadcast_to(v.reshape(1, -1), (8, h)).astype(jnp.float32)


def kernel(x, edge_index, W1, b1, g1, be1, W2, b2, g2, be2, W3, b3, g3, be3,
           W4, b4, g4, be4, Wc, bc):
    src = edge_index[0]
    dst = edge_index[1]

    counts = _make_hist()(dst)
    s_arr, meta = _scan(counts)
    ssrc, sdloc = _make_scatter()(src, dst, s_arr, meta)
    degb = _make_deg()(sdloc, meta)
    delta = _prep(degb)

    x80 = jnp.pad(x, ((0, NPAD - N), (0, 0))).reshape(NOWN, RPO, 128)
    x80 = jnp.pad(x80, ((0, 0), (0, 1), (0, 0))).reshape(NP80, 128)

    r1 = _make_reduce(NP80, 128, 2, 128)(x80, ssrc, sdloc, meta)
    y1, st1 = _make_layer(128, 128)(*r1, degb, delta, W1, _b8(b1, 128))
    h1 = _make_bn(128, False)(y1, st1, _b8(g1, 128), _b8(be1, 128))

    r2 = _make_reduce(NP80, 128, 2, 128)(h1, ssrc, sdloc, meta)
    y2, st2 = _make_layer(128, 256)(*r2, degb, delta, W2, _b8(b2, 256))
    h2 = _make_bn(256, False)(y2, st2, _b8(g2, 256), _b8(be2, 256))

    r3 = _make_reduce(NP80, 256, 1, 64)(h2, ssrc, sdloc, meta)
    y3, st3 = _make_layer(256, 128)(*r3, degb, delta, W3, _b8(b3, 128))
    h3 = _make_bn(128, False)(y3, st3, _b8(g3, 128), _b8(be3, 128))

    w4p = jnp.pad(W4, ((0, 0), (0, 64)))
    b4p = jnp.pad(b4, (0, 64))
    g4p = jnp.pad(g4, (0, 64))
    be4p = jnp.pad(be4, (0, 64))
    wcp = jnp.pad(Wc, ((0, 64), (0, 112)))
    bcp = jnp.pad(bc, (0, 112))
    r4 = _make_reduce(NP80, 128, 2, 128)(h3, ssrc, sdloc, meta)
    y4, st4 = _make_layer(128, 128)(*r4, degb, delta, w4p, _b8(b4p, 128))
    out = _make_bn(128, True)(y4, st4, _b8(g4p, 128), _b8(be4p, 128),
                              wcp, _b8(bcp, 128))
    out = out.reshape(NOWN, STRIDE, 128)[:, :RPO, :].reshape(NPAD, 128)
    return out[:N, :16]
